# baseline (speedup 1.0000x reference)
import jax
import jax.numpy as jnp
from jax.experimental import pallas as pl
from jax.experimental.pallas import tpu as pltpu

TOP_K = 8
NUM_EXPERTS = 64
HIDDEN = 768
TOKENS = 32768
BT = 4096


def _router_block(hs1_ref, hs2_ref, gw_ref, w_ref, i_ref):
    dn = (((1,), (1,)), ((), ()))
    lt = jax.lax.dot_general(
        gw_ref[:, :HIDDEN // 2], hs1_ref[...], dimension_numbers=dn,
        preferred_element_type=jnp.float32,
    ) + jax.lax.dot_general(
        gw_ref[:, HIDDEN // 2:], hs2_ref[...], dimension_numbers=dn,
        preferred_element_type=jnp.float32,
    )
    row = jax.lax.broadcasted_iota(jnp.int32, (NUM_EXPERTS, BT), 0)
    x = lt
    neg_inf = jnp.float32(-jnp.inf)
    vals = []
    idxs = []
    for _ in range(TOP_K):
        m = jnp.max(x, axis=0, keepdims=True)
        hit = x >= m
        a = jnp.min(jnp.where(hit, row, NUM_EXPERTS), axis=0,
                    keepdims=True)
        vals.append(m)
        idxs.append(a)
        x = jnp.where(row == a, neg_inf, x)
    v = jnp.concatenate(vals, axis=0)
    e = jnp.exp(v - v[0:1, :])
    w_ref[...] = e / jnp.sum(e, axis=0, keepdims=True)
    i_ref[...] = jnp.concatenate(idxs, axis=0)


@jax.jit
def kernel(hidden_states, gate_weight):
    grid = (TOKENS // BT,)
    w, i = pl.pallas_call(
        _router_block,
        grid=grid,
        in_specs=[
            pl.BlockSpec((BT, HIDDEN // 2), lambda t: (t, 0)),
            pl.BlockSpec((BT, HIDDEN // 2), lambda t: (t, 1)),
            pl.BlockSpec((NUM_EXPERTS, HIDDEN), lambda t: (0, 0)),
        ],
        out_specs=[
            pl.BlockSpec((TOP_K, BT), lambda t: (0, t)),
            pl.BlockSpec((TOP_K, BT), lambda t: (0, t)),
        ],
        out_shape=[
            jax.ShapeDtypeStruct((TOP_K, TOKENS), jnp.float32),
            jax.ShapeDtypeStruct((TOP_K, TOKENS), jnp.int32),
        ],
        compiler_params=pltpu.CompilerParams(
            dimension_semantics=("parallel",),
        ),
    )(hidden_states, hidden_states, gate_weight)
    return (w.T, i.T)


# speedup vs baseline: 1.0657x; 1.0657x over previous
import jax
import jax.numpy as jnp
from jax.experimental import pallas as pl
from jax.experimental.pallas import tpu as pltpu

TOP_K = 8
NUM_EXPERTS = 64
HIDDEN = 768
TOKENS = 32768
BT = 4096


def _router_block(hs_ref, gw_ref, w_ref, i_ref):
    lt = jax.lax.dot_general(
        gw_ref[...], hs_ref[...],
        dimension_numbers=(((1,), (1,)), ((), ())),
        preferred_element_type=jnp.float32,
    )
    row = jax.lax.broadcasted_iota(jnp.int32, (NUM_EXPERTS, BT), 0)
    x = lt
    neg_inf = jnp.float32(-jnp.inf)
    vals = []
    idxs = []
    for _ in range(TOP_K):
        m = jnp.max(x, axis=0, keepdims=True)
        hit = x >= m
        a = jnp.min(jnp.where(hit, row, NUM_EXPERTS), axis=0,
                    keepdims=True)
        vals.append(m)
        idxs.append(a)
        x = jnp.where(row == a, neg_inf, x)
    v = jnp.concatenate(vals, axis=0)
    e = jnp.exp(v - v[0:1, :])
    w_ref[...] = e / jnp.sum(e, axis=0, keepdims=True)
    i_ref[...] = jnp.concatenate(idxs, axis=0)


@jax.jit
def kernel(hidden_states, gate_weight):
    grid = (TOKENS // BT,)
    w, i = pl.pallas_call(
        _router_block,
        grid=grid,
        in_specs=[
            pl.BlockSpec((BT, HIDDEN), lambda t: (t, 0)),
            pl.BlockSpec((NUM_EXPERTS, HIDDEN), lambda t: (0, 0)),
        ],
        out_specs=[
            pl.BlockSpec((TOP_K, BT), lambda t: (0, t)),
            pl.BlockSpec((TOP_K, BT), lambda t: (0, t)),
        ],
        out_shape=[
            jax.ShapeDtypeStruct((TOP_K, TOKENS), jnp.float32),
            jax.ShapeDtypeStruct((TOP_K, TOKENS), jnp.int32),
        ],
        compiler_params=pltpu.CompilerParams(
            dimension_semantics=("arbitrary",),
        ),
    )(hidden_states, gate_weight)
    return (w.T, i.T)
